# baseline (device time: 39847 ns/iter reference)
import jax
import jax.numpy as jnp
from jax import lax
from jax.experimental import pallas as pl
from jax.experimental.pallas import tpu as pltpu

N_DEV = 4
M = 4096
K_LOC = 1024
M_LOC = 1024
N = 2048

NCHUNK = 8
CH = M_LOC // NCHUNK

FP8 = jnp.float8_e4m3fn

SEND_ORDER = (1, 3, 2)
W_ORDER = (0, 1, 3, 2)
SEM_BASE = {1: 0, 3: NCHUNK, 2: 2 * NCHUNK}


def kernel(x, w_mat, scale_x, scale_w):

    def body(x_hbm, w_hbm, sx_ref, sw_ref, out_hbm,
             xstage_ref, x8s_ref, xfull_ref, wstage_ref, w8_ref, ovmem_ref,
             xload_sems, wcopy_sems, ocopy_sems, send_sems, recv_sems):
        my = lax.axis_index("i")

        def start_xload(block_idx, slot):
            d = SEND_ORDER[block_idx] if block_idx < 3 else 0
            peer = lax.rem(my + d, N_DEV)
            cp = pltpu.make_async_copy(
                x_hbm.at[pl.ds(peer * M_LOC, M_LOC), :],
                xstage_ref.at[slot],
                xload_sems.at[slot],
            )
            cp.start()
            return cp

        def start_wload(d, slot):
            src = lax.rem(my - d + N_DEV, N_DEV)
            cp = pltpu.make_async_copy(
                w_hbm.at[pl.ds(src * K_LOC, K_LOC), :],
                wstage_ref.at[slot],
                wcopy_sems.at[slot],
            )
            cp.start()
            return cp

        def chunk_copy(o, d, j):
            peer = lax.rem(my + d, N_DEV)
            return pltpu.make_async_remote_copy(
                src_ref=x8s_ref.at[o, pl.ds(j * CH, CH), :],
                dst_ref=xfull_ref.at[d, pl.ds(j * CH, CH), :],
                send_sem=send_sems.at[SEM_BASE[d] + j],
                recv_sem=recv_sems.at[SEM_BASE[d] + j],
                device_id=(peer,),
                device_id_type=pl.DeviceIdType.MESH,
            )

        xl0 = start_xload(0, 0)
        xl1 = start_xload(1, 1)

        barrier = pltpu.get_barrier_semaphore()
        for d in (1, 2, 3):
            peer = lax.rem(my + d, N_DEV)
            pl.semaphore_signal(
                barrier, inc=1,
                device_id=(peer,), device_id_type=pl.DeviceIdType.MESH,
            )
        pl.semaphore_wait(barrier, 3)

        rchunks = {}

        def fire(o, d):
            x8s_ref[o] = xstage_ref[o % 2].astype(FP8)
            for j in range(NCHUNK):
                rc = chunk_copy(o, d, j)
                rc.start()
                rchunks[(d, j)] = rc

        xl0.wait()
        fire(0, 1)
        xl2 = start_xload(2, 0)

        xl1.wait()
        fire(1, 3)
        xl3 = start_xload(3, 1)

        xl2.wait()
        fire(2, 2)

        wl = {W_ORDER[0]: start_wload(W_ORDER[0], 0)}

        xl3.wait()
        xfull_ref[0] = xstage_ref[1].astype(FP8)

        wl[W_ORDER[1]] = start_wload(W_ORDER[1], 1)

        def finish_w(i):
            d = W_ORDER[i]
            wl[d].wait()
            w8_ref[d] = wstage_ref[i % 2].astype(FP8)
            if i + 2 < N_DEV:
                nxt = W_ORDER[i + 2]
                wl[nxt] = start_wload(nxt, i % 2)

        finish_w(0)
        ovmem_ref[...] = jnp.dot(
            xfull_ref[0], w8_ref[0], preferred_element_type=jnp.float32
        )
        finish_w(1)
        finish_w(2)
        finish_w(3)

        scale = sx_ref[0] * sw_ref[0]
        ocopies = []
        for j in range(NCHUNK):
            rows = pl.ds(j * CH, CH)
            rchunks[(1, j)].wait_recv()
            rchunks[(3, j)].wait_recv()
            rchunks[(2, j)].wait_recv()
            ovmem_ref[rows, :] = jnp.maximum(
                (ovmem_ref[rows, :]
                 + jnp.dot(xfull_ref[1, rows, :], w8_ref[1],
                           preferred_element_type=jnp.float32)
                 + jnp.dot(xfull_ref[3, rows, :], w8_ref[3],
                           preferred_element_type=jnp.float32)
                 + jnp.dot(xfull_ref[2, rows, :], w8_ref[2],
                           preferred_element_type=jnp.float32)) * scale,
                0.0,
            )
            ocp = pltpu.make_async_copy(
                ovmem_ref.at[rows, :], out_hbm.at[rows, :], ocopy_sems.at[j],
            )
            ocp.start()
            ocopies.append(ocp)

        for ocp in ocopies:
            ocp.wait()
        for rc in rchunks.values():
            rc.wait_send()

    return pl.pallas_call(
        body,
        out_shape=jax.ShapeDtypeStruct((M_LOC, N), jnp.float32),
        in_specs=[
            pl.BlockSpec(memory_space=pl.ANY),
            pl.BlockSpec(memory_space=pl.ANY),
            pl.BlockSpec(memory_space=pltpu.SMEM),
            pl.BlockSpec(memory_space=pltpu.SMEM),
        ],
        out_specs=pl.BlockSpec(memory_space=pl.ANY),
        scratch_shapes=[
            pltpu.VMEM((2, M_LOC, K_LOC), jnp.float32),
            pltpu.VMEM((3, M_LOC, K_LOC), FP8),
            pltpu.VMEM((N_DEV, M_LOC, K_LOC), FP8),
            pltpu.VMEM((2, K_LOC, N), jnp.float32),
            pltpu.VMEM((N_DEV, K_LOC, N), FP8),
            pltpu.VMEM((M_LOC, N), jnp.float32),
            pltpu.SemaphoreType.DMA((2,)),
            pltpu.SemaphoreType.DMA((2,)),
            pltpu.SemaphoreType.DMA((NCHUNK,)),
            pltpu.SemaphoreType.DMA((3 * NCHUNK,)),
            pltpu.SemaphoreType.DMA((3 * NCHUNK,)),
        ],
        compiler_params=pltpu.CompilerParams(
            collective_id=0,
            vmem_limit_bytes=60 * 1024 * 1024,
        ),
    )(x, w_mat, scale_x, scale_w)


# device time: 39560 ns/iter; 1.0073x vs baseline; 1.0073x over previous
import jax
import jax.numpy as jnp
from jax import lax
from jax.experimental import pallas as pl
from jax.experimental.pallas import tpu as pltpu

N_DEV = 4
M = 4096
K_LOC = 1024
M_LOC = 1024
N = 2048

NCHUNK = 4
CH = M_LOC // NCHUNK

FP8 = jnp.float8_e4m3fn

SEND_ORDER = (1, 3, 2)
W_ORDER = (0, 1, 3, 2)
SEM_BASE = {1: 0, 3: NCHUNK, 2: 2 * NCHUNK}


def kernel(x, w_mat, scale_x, scale_w):

    def body(x_hbm, w_hbm, sx_ref, sw_ref, out_hbm,
             xstage_ref, x8s_ref, xfull_ref, wstage_ref, w8_ref, ovmem_ref,
             xload_sems, wcopy_sems, ocopy_sems, send_sems, recv_sems):
        my = lax.axis_index("i")

        def start_xload(block_idx, slot):
            d = SEND_ORDER[block_idx] if block_idx < 3 else 0
            peer = lax.rem(my + d, N_DEV)
            cp = pltpu.make_async_copy(
                x_hbm.at[pl.ds(peer * M_LOC, M_LOC), :],
                xstage_ref.at[slot],
                xload_sems.at[slot],
            )
            cp.start()
            return cp

        def start_wload(d, slot):
            src = lax.rem(my - d + N_DEV, N_DEV)
            cp = pltpu.make_async_copy(
                w_hbm.at[pl.ds(src * K_LOC, K_LOC), :],
                wstage_ref.at[slot],
                wcopy_sems.at[slot],
            )
            cp.start()
            return cp

        def chunk_copy(o, d, j):
            peer = lax.rem(my + d, N_DEV)
            return pltpu.make_async_remote_copy(
                src_ref=x8s_ref.at[o, pl.ds(j * CH, CH), :],
                dst_ref=xfull_ref.at[d, pl.ds(j * CH, CH), :],
                send_sem=send_sems.at[SEM_BASE[d] + j],
                recv_sem=recv_sems.at[SEM_BASE[d] + j],
                device_id=(peer,),
                device_id_type=pl.DeviceIdType.MESH,
            )

        xl0 = start_xload(0, 0)
        xl1 = start_xload(1, 1)

        barrier = pltpu.get_barrier_semaphore()
        for d in (1, 2, 3):
            peer = lax.rem(my + d, N_DEV)
            pl.semaphore_signal(
                barrier, inc=1,
                device_id=(peer,), device_id_type=pl.DeviceIdType.MESH,
            )
        pl.semaphore_wait(barrier, 3)

        rchunks = {}

        def fire(o, d):
            x8s_ref[o] = xstage_ref[o % 2].astype(FP8)
            for j in range(NCHUNK):
                rc = chunk_copy(o, d, j)
                rc.start()
                rchunks[(d, j)] = rc

        xl0.wait()
        fire(0, 1)
        xl2 = start_xload(2, 0)

        xl1.wait()
        fire(1, 3)
        xl3 = start_xload(3, 1)

        xl2.wait()
        fire(2, 2)

        wl = {W_ORDER[0]: start_wload(W_ORDER[0], 0)}

        xl3.wait()
        xfull_ref[0] = xstage_ref[1].astype(FP8)

        wl[W_ORDER[1]] = start_wload(W_ORDER[1], 1)

        def finish_w(i):
            d = W_ORDER[i]
            wl[d].wait()
            w8_ref[d] = wstage_ref[i % 2].astype(FP8)
            if i + 2 < N_DEV:
                nxt = W_ORDER[i + 2]
                wl[nxt] = start_wload(nxt, i % 2)

        finish_w(0)
        ovmem_ref[...] = jnp.dot(
            xfull_ref[0], w8_ref[0], preferred_element_type=jnp.float32
        )
        finish_w(1)
        finish_w(2)
        finish_w(3)

        scale = sx_ref[0] * sw_ref[0]
        ocopies = []
        for j in range(NCHUNK):
            rows = pl.ds(j * CH, CH)
            rchunks[(1, j)].wait_recv()
            ovmem_ref[rows, :] += jnp.dot(
                xfull_ref[1, rows, :], w8_ref[1],
                preferred_element_type=jnp.float32,
            )
            rchunks[(3, j)].wait_recv()
            ovmem_ref[rows, :] += jnp.dot(
                xfull_ref[3, rows, :], w8_ref[3],
                preferred_element_type=jnp.float32,
            )
            rchunks[(2, j)].wait_recv()
            ovmem_ref[rows, :] = jnp.maximum(
                (ovmem_ref[rows, :]
                 + jnp.dot(xfull_ref[2, rows, :], w8_ref[2],
                           preferred_element_type=jnp.float32)) * scale,
                0.0,
            )
            ocp = pltpu.make_async_copy(
                ovmem_ref.at[rows, :], out_hbm.at[rows, :], ocopy_sems.at[j],
            )
            ocp.start()
            ocopies.append(ocp)

        for ocp in ocopies:
            ocp.wait()
        for rc in rchunks.values():
            rc.wait_send()

    return pl.pallas_call(
        body,
        out_shape=jax.ShapeDtypeStruct((M_LOC, N), jnp.float32),
        in_specs=[
            pl.BlockSpec(memory_space=pl.ANY),
            pl.BlockSpec(memory_space=pl.ANY),
            pl.BlockSpec(memory_space=pltpu.SMEM),
            pl.BlockSpec(memory_space=pltpu.SMEM),
        ],
        out_specs=pl.BlockSpec(memory_space=pl.ANY),
        scratch_shapes=[
            pltpu.VMEM((2, M_LOC, K_LOC), jnp.float32),
            pltpu.VMEM((3, M_LOC, K_LOC), FP8),
            pltpu.VMEM((N_DEV, M_LOC, K_LOC), FP8),
            pltpu.VMEM((2, K_LOC, N), jnp.float32),
            pltpu.VMEM((N_DEV, K_LOC, N), FP8),
            pltpu.VMEM((M_LOC, N), jnp.float32),
            pltpu.SemaphoreType.DMA((2,)),
            pltpu.SemaphoreType.DMA((2,)),
            pltpu.SemaphoreType.DMA((NCHUNK,)),
            pltpu.SemaphoreType.DMA((3 * NCHUNK,)),
            pltpu.SemaphoreType.DMA((3 * NCHUNK,)),
        ],
        compiler_params=pltpu.CompilerParams(
            collective_id=0,
            vmem_limit_bytes=60 * 1024 * 1024,
        ),
    )(x, w_mat, scale_x, scale_w)
